# baseline (speedup 1.0000x reference)
import jax
import jax.numpy as jnp
from jax.experimental import pallas as pl
from jax.experimental.pallas import tpu as pltpu

N_EMB = 1024
EMB_DIM = 64
BETA = 0.25
N_ROWS = 8192
BLK = 2048
GRID = N_ROWS // BLK


def _tc_body(z_ref, embt_ref, emb_ref,
             dist_ref, idx_ref, zq_ref, counts_ref, loss_ref, perp_ref,
             minsum_ref):
    i = pl.program_id(0)
    z = z_ref[...]
    et = embt_ref[...]
    z2 = jnp.sum(z * z, axis=1, keepdims=True)
    e2 = jnp.sum(et * et, axis=0, keepdims=True)
    d = (z2 + e2) - jnp.dot(z + z, et, preferred_element_type=jnp.float32)
    dist_ref[...] = d

    mind = jnp.min(d, axis=1)
    iotaf = jax.lax.broadcasted_iota(
        jnp.int32, (BLK, N_EMB), 1).astype(jnp.float32)
    idxf = jnp.min(jnp.where(d == mind[:, None], iotaf, float(N_EMB)),
                   axis=1)
    idx = idxf.astype(jnp.int32)
    idx_ref[...] = idx.reshape(1, 1, BLK)

    onehot = (iotaf == idxf[:, None]).astype(jnp.float32)
    zq_ref[...] = jnp.dot(onehot, emb_ref[...],
                          preferred_element_type=jnp.float32)
    cnt = jnp.sum(onehot, axis=0, keepdims=True)

    @pl.when(i == 0)
    def _init():
        counts_ref[...] = jnp.zeros((1, N_EMB), jnp.float32)
        minsum_ref[0, 0] = 0.0

    counts_ref[...] += cnt
    minsum_ref[0, 0] += jnp.sum(mind)

    @pl.when(i == GRID - 1)
    def _final():
        loss = BETA * minsum_ref[0, 0] / float(N_ROWS * EMB_DIM)
        loss_ref[...] = jnp.full((1, 1), loss, jnp.float32)
        p = counts_ref[...] / float(N_ROWS)
        perp = jnp.exp(-jnp.sum(p * jnp.log(p + 1e-10)))
        perp_ref[...] = jnp.full((1, 1), perp, jnp.float32)


def kernel(z_e, embedding):
    B, D, H, W = z_e.shape
    z_flat = jnp.transpose(z_e, (0, 2, 3, 1)).reshape(N_ROWS, EMB_DIM)
    emb_t = embedding.T

    out_shapes = (
        jax.ShapeDtypeStruct((N_ROWS, N_EMB), jnp.float32),
        jax.ShapeDtypeStruct((GRID, 1, BLK), jnp.int32),
        jax.ShapeDtypeStruct((N_ROWS, EMB_DIM), jnp.float32),
        jax.ShapeDtypeStruct((1, N_EMB), jnp.float32),
        jax.ShapeDtypeStruct((1, 1), jnp.float32),
        jax.ShapeDtypeStruct((1, 1), jnp.float32),
    )
    dist, idx3, zq, counts, loss, perp = pl.pallas_call(
        _tc_body,
        grid=(GRID,),
        in_specs=[
            pl.BlockSpec((BLK, EMB_DIM), lambda i: (i, 0)),
            pl.BlockSpec((EMB_DIM, N_EMB), lambda i: (0, 0)),
            pl.BlockSpec((N_EMB, EMB_DIM), lambda i: (0, 0)),
        ],
        out_specs=(
            pl.BlockSpec((BLK, N_EMB), lambda i: (i, 0)),
            pl.BlockSpec((1, 1, BLK), lambda i: (i, 0, 0)),
            pl.BlockSpec((BLK, EMB_DIM), lambda i: (i, 0)),
            pl.BlockSpec((1, N_EMB), lambda i: (0, 0)),
            pl.BlockSpec((1, 1), lambda i: (0, 0)),
            pl.BlockSpec((1, 1), lambda i: (0, 0)),
        ),
        out_shape=out_shapes,
        scratch_shapes=[pltpu.SMEM((1, 1), jnp.float32)],
    )(z_flat, emb_t, embedding)

    encoding_indices = idx3.reshape(N_ROWS)
    z_q_out = jnp.transpose(zq.reshape(B, H, W, D), (0, 3, 1, 2))
    return (z_q_out, loss.reshape(()), perp.reshape(()),
            encoding_indices, dist)
